# baseline (speedup 1.0000x reference)
import functools

import jax
import jax.numpy as jnp
from jax import lax
from jax.experimental import pallas as pl
from jax.experimental.pallas import tpu as pltpu
from jax.experimental.pallas import tpu_sc as plsc

N = 10000
NP = 10240
DIN = 256
C = 40
E = 160000

NC = 2
NS = 16
NW = NC * NS
K = 128
EPT = 5376
NCAP = EPT // K
E2P = NW * EPT
NR = 4
RSPAN = NP // NR
AROWS = RSPAN + 128
ARPT = AROWS // NS
JUNK_DST = RSPAN
BN = 512

_SC_MESH = plsc.VectorSubcoreMesh(core_axis_name="c", subcore_axis_name="s")
_SC_CP = pltpu.CompilerParams(needs_layout_passes=False)


def _deg_body(dst_hbm, src_hbm, deg_hbm, lsrc_hbm, ldst_hbm, cnt_hbm,
              dst_v, src_v, hist, lsrc, ldst, cbuf):
    cid = lax.axis_index("c")
    sid = lax.axis_index("s")
    wid = cid * NS + sid
    pltpu.sync_copy(dst_hbm.at[wid], dst_v)
    pltpu.sync_copy(src_hbm.at[wid], src_v)

    @pl.loop(0, NP, step=16)
    def _(i):
        hist[pl.ds(i, 16)] = jnp.zeros((16,), jnp.float32)

    ones = jnp.ones((16,), jnp.float32)

    @pl.loop(0, EPT, step=16)
    def _(e):
        plsc.addupdate_scatter(hist, [dst_v[pl.ds(e, 16)]], ones)

    pltpu.sync_copy(hist, deg_hbm.at[wid])

    junk_s = jnp.full((16,), NP - 1, jnp.int32)
    junk_d = jnp.full((16,), JUNK_DST, jnp.int32)

    @pl.loop(0, NR * EPT, step=16)
    def _(e):
        lsrc[pl.ds(e, 16)] = junk_s
        ldst[pl.ds(e, 16)] = junk_d

    def body(i, bases):
        d16 = dst_v[pl.ds(i * 16, 16)]
        s16 = src_v[pl.ds(i * 16, 16)]
        q16 = d16 // RSPAN
        new_bases = []
        for q in range(NR):
            m = q16 == q
            mi = m.astype(jnp.int32)
            pos = bases[q] + plsc.cumsum(mi) - 1 + q * EPT
            plsc.store_scatter(lsrc, [pos], s16, mask=m)
            plsc.store_scatter(ldst, [pos], d16 - q * RSPAN, mask=m)
            new_bases.append(bases[q] + plsc.all_reduce_population_count(m))
        return tuple(new_bases)

    z16 = jnp.zeros((16,), jnp.int32)
    bases = lax.fori_loop(0, EPT // 16, body, (z16, z16, z16, z16))

    for q in range(NR):
        cbuf[pl.ds(q * 16, 16)] = bases[q]
    pltpu.sync_copy(cbuf, cnt_hbm.at[wid])
    pltpu.sync_copy(lsrc, lsrc_hbm.at[wid])
    pltpu.sync_copy(ldst, ldst_hbm.at[wid])


_deg_kernel = functools.partial(
    pl.kernel,
    out_type=(jax.ShapeDtypeStruct((NW, NP), jnp.float32),
              jax.ShapeDtypeStruct((NW, NR * EPT), jnp.int32),
              jax.ShapeDtypeStruct((NW, NR * EPT), jnp.int32),
              jax.ShapeDtypeStruct((NW, NR * 16), jnp.int32)),
    mesh=_SC_MESH,
    compiler_params=_SC_CP,
    scratch_types=[
        pltpu.VMEM((EPT,), jnp.int32),
        pltpu.VMEM((EPT,), jnp.int32),
        pltpu.VMEM((NP,), jnp.float32),
        pltpu.VMEM((NR * EPT,), jnp.int32),
        pltpu.VMEM((NR * EPT,), jnp.int32),
        pltpu.VMEM((NR * 16,), jnp.int32),
    ],
)(_deg_body)


def _msg_body(hw_hbm, lsrc_hbm, ldst_hbm, cnt_hbm, out_hbm,
              sidx, didx, cvec, buf, zbuf, agg):
    cid = lax.axis_index("c")
    sid = lax.axis_index("s")

    @pl.loop(0, 8)
    def _(r):
        for hh in (0, 1):
            @pl.loop(0, 128, step=16)
            def _(j):
                zbuf[r, hh, pl.ds(j, 16)] = jnp.zeros((16,), jnp.float32)

    for qi in (0, 1):
        q = cid * 2 + qi

        @pl.loop(0, ARPT, step=8)
        def _(r):
            pltpu.sync_copy(zbuf, agg.at[pl.ds(sid * ARPT + r, 8)])

        plsc.subcore_barrier()

        for sk in (0, 1):
            slot = (sk * NS + sid) * NR + q
            pltpu.sync_copy(cnt_hbm.at[slot], cvec)
            cnt = jnp.max(cvec[...])
            nch = (cnt + (K - 1)) // K
            pltpu.sync_copy(lsrc_hbm.at[slot], sidx)
            pltpu.sync_copy(ldst_hbm.at[slot], didx)

            @pl.loop(0, NCAP)
            def _(c):
                @pl.when(c < nch)
                def _():
                    pltpu.sync_copy(hw_hbm.at[sidx.at[c]], buf)
                    pltpu.sync_copy(buf, agg.at[didx.at[c]], add=True)

        plsc.subcore_barrier()
        pltpu.sync_copy(agg.at[pl.ds(sid * ARPT, ARPT)],
                        out_hbm.at[q, pl.ds(sid * ARPT, ARPT)])

        plsc.subcore_barrier()


_msg_kernel = functools.partial(
    pl.kernel,
    out_type=jax.ShapeDtypeStruct((NR, AROWS, 2, 128), jnp.float32),
    mesh=_SC_MESH,
    compiler_params=_SC_CP,
    scratch_types=[
        pltpu.VMEM((NCAP, K), jnp.int32),
        pltpu.VMEM((NCAP, K), jnp.int32),
        pltpu.VMEM((16,), jnp.int32),
        pltpu.VMEM((K, 2, 128), jnp.float32),
        pltpu.VMEM((8, 2, 128), jnp.float32),
        pltpu.VMEM_SHARED((AROWS, 2, 128), jnp.float32),
    ],
)(_msg_body)


def _dense_body(x_ref, w1_ref, b1_ref, wg_ref, deg_ref, hw_ref, dinv_ref):
    h = jnp.maximum(
        jnp.dot(x_ref[...], w1_ref[...],
                preferred_element_type=jnp.float32) + b1_ref[...], 0.0)
    hw = jnp.dot(h, wg_ref[...], preferred_element_type=jnp.float32)
    deg = jnp.sum(deg_ref[...], axis=0)
    dinv = jnp.where(deg > 0, lax.rsqrt(deg), 0.0)
    dinv_ref[...] = dinv[None, :]
    hw_ref[...] = hw * dinv[:, None]


def _final_body(p_ref, dinv_ref, bg_ref, wc_ref, bc_ref, o_ref):
    dinv = dinv_ref[0, :][:, None]
    agg = p_ref[0].reshape(BN, DIN)
    h2 = jnp.maximum(agg * dinv + bg_ref[...], 0.0)
    o_ref[...] = jnp.dot(h2, wc_ref[...],
                         preferred_element_type=jnp.float32) + bc_ref[...]


def kernel(X, edge_index, W1, b1, Wg, bg, Wc, bc):
    src = edge_index[0].astype(jnp.int32)
    dst = edge_index[1].astype(jnp.int32)
    loop_idx = jnp.arange(N, dtype=jnp.int32)
    pad_idx = jnp.full((E2P - E - N,), NP - 1, dtype=jnp.int32)
    srcf = jnp.concatenate([src, loop_idx, pad_idx]).reshape(NW, EPT)
    dstf = jnp.concatenate([dst, loop_idx, pad_idx]).reshape(NW, EPT)
    Xp = jnp.pad(X, ((0, NP - N), (0, 0)))

    degp, lsrc, ldst, cnts = _deg_kernel(dstf, srcf)

    hw2, dinv = pl.pallas_call(
        _dense_body,
        grid=(NP // BN,),
        in_specs=[
            pl.BlockSpec((BN, DIN), lambda i: (i, 0)),
            pl.BlockSpec((DIN, DIN), lambda i: (0, 0)),
            pl.BlockSpec((1, DIN), lambda i: (0, 0)),
            pl.BlockSpec((DIN, DIN), lambda i: (0, 0)),
            pl.BlockSpec((NW, BN), lambda i: (0, i)),
        ],
        out_specs=[
            pl.BlockSpec((BN, DIN), lambda i: (i, 0)),
            pl.BlockSpec((1, BN), lambda i: (0, i)),
        ],
        out_shape=[
            jax.ShapeDtypeStruct((NP, DIN), jnp.float32),
            jax.ShapeDtypeStruct((1, NP), jnp.float32),
        ],
    )(Xp, W1, b1.reshape(1, DIN), Wg, degp)

    p = _msg_kernel(hw2.reshape(NP, 2, 128),
                    lsrc.reshape(NW * NR, NCAP, K),
                    ldst.reshape(NW * NR, NCAP, K),
                    cnts.reshape(NW * NR, 16))

    out = pl.pallas_call(
        _final_body,
        grid=(NP // BN,),
        in_specs=[
            pl.BlockSpec((1, BN, 2, 128), lambda i: (i // 5, i % 5, 0, 0)),
            pl.BlockSpec((1, BN), lambda i: (0, i)),
            pl.BlockSpec((1, DIN), lambda i: (0, 0)),
            pl.BlockSpec((DIN, C), lambda i: (0, 0)),
            pl.BlockSpec((1, C), lambda i: (0, 0)),
        ],
        out_specs=pl.BlockSpec((BN, C), lambda i: (i, 0)),
        out_shape=jax.ShapeDtypeStruct((N, C), jnp.float32),
    )(p, dinv, bg.reshape(1, DIN), Wc, bc.reshape(1, C))

    return out


# speedup vs baseline: 1.0689x; 1.0689x over previous
import functools

import jax
import jax.numpy as jnp
from jax import lax
from jax.experimental import pallas as pl
from jax.experimental.pallas import tpu as pltpu
from jax.experimental.pallas import tpu_sc as plsc

N = 10000
NP = 10240
DIN = 256
C = 40
E = 160000

NC = 2
NS = 16
NW = NC * NS
K = 128
EPT = 5376
NCAP = EPT // K
E2P = NW * EPT
NR = 4
RSPAN = NP // NR
AROWS = RSPAN + 128
ARPT = AROWS // NS
JUNK_DST = RSPAN
BN = 512

_SC_MESH = plsc.VectorSubcoreMesh(core_axis_name="c", subcore_axis_name="s")
_SC_CP = pltpu.CompilerParams(needs_layout_passes=False)


def _deg_body(dst_hbm, src_hbm, deg_hbm, lsrc_hbm, ldst_hbm, cnt_hbm,
              dst_v, src_v, hist, lsrc, ldst, cbuf):
    cid = lax.axis_index("c")
    sid = lax.axis_index("s")
    wid = cid * NS + sid
    pltpu.sync_copy(dst_hbm.at[wid], dst_v)
    pltpu.sync_copy(src_hbm.at[wid], src_v)

    @pl.loop(0, NP, step=16)
    def _(i):
        hist[pl.ds(i, 16)] = jnp.zeros((16,), jnp.float32)

    ones = jnp.ones((16,), jnp.float32)

    @pl.loop(0, EPT, step=16)
    def _(e):
        plsc.addupdate_scatter(hist, [dst_v[pl.ds(e, 16)]], ones)

    pltpu.sync_copy(hist, deg_hbm.at[wid])

    junk_s = jnp.full((16,), NP - 1, jnp.int32)
    junk_d = jnp.full((16,), JUNK_DST, jnp.int32)

    @pl.loop(0, NR * EPT, step=16)
    def _(e):
        lsrc[pl.ds(e, 16)] = junk_s
        ldst[pl.ds(e, 16)] = junk_d

    def body(i, bases):
        d16 = dst_v[pl.ds(i * 16, 16)]
        s16 = src_v[pl.ds(i * 16, 16)]
        q16 = d16 // RSPAN
        new_bases = []
        for q in range(NR):
            m = q16 == q
            mi = m.astype(jnp.int32)
            pos = bases[q] + plsc.cumsum(mi) - 1 + q * EPT
            plsc.store_scatter(lsrc, [pos], s16, mask=m)
            plsc.store_scatter(ldst, [pos], d16 - q * RSPAN, mask=m)
            new_bases.append(bases[q] + plsc.all_reduce_population_count(m))
        return tuple(new_bases)

    z16 = jnp.zeros((16,), jnp.int32)
    bases = lax.fori_loop(0, EPT // 16, body, (z16, z16, z16, z16))

    for q in range(NR):
        cbuf[pl.ds(q * 16, 16)] = bases[q]
    pltpu.sync_copy(cbuf, cnt_hbm.at[wid])
    pltpu.sync_copy(lsrc, lsrc_hbm.at[wid])
    pltpu.sync_copy(ldst, ldst_hbm.at[wid])


_deg_kernel = functools.partial(
    pl.kernel,
    out_type=(jax.ShapeDtypeStruct((NW, NP), jnp.float32),
              jax.ShapeDtypeStruct((NW, NR * EPT), jnp.int32),
              jax.ShapeDtypeStruct((NW, NR * EPT), jnp.int32),
              jax.ShapeDtypeStruct((NW, NR * 16), jnp.int32)),
    mesh=_SC_MESH,
    compiler_params=_SC_CP,
    scratch_types=[
        pltpu.VMEM((EPT,), jnp.int32),
        pltpu.VMEM((EPT,), jnp.int32),
        pltpu.VMEM((NP,), jnp.float32),
        pltpu.VMEM((NR * EPT,), jnp.int32),
        pltpu.VMEM((NR * EPT,), jnp.int32),
        pltpu.VMEM((NR * 16,), jnp.int32),
    ],
)(_deg_body)


def _msg_body(hw_hbm, lsrc_hbm, ldst_hbm, cnt_hbm, out_hbm,
              sidx, didx, cvec, buf0, buf1, zbuf, agg, gs0, gs1, ss0, ss1):
    cid = lax.axis_index("c")
    sid = lax.axis_index("s")

    @pl.loop(0, 8)
    def _(r):
        for hh in (0, 1):
            @pl.loop(0, 128, step=16)
            def _(j):
                zbuf[r, hh, pl.ds(j, 16)] = jnp.zeros((16,), jnp.float32)

    for qi in (0, 1):
        q = cid * 2 + qi

        @pl.loop(0, ARPT, step=8)
        def _(r):
            pltpu.sync_copy(zbuf, agg.at[pl.ds(sid * ARPT + r, 8)])

        plsc.subcore_barrier()

        for sk in (0, 1):
            slot = (sk * NS + sid) * NR + q
            pltpu.sync_copy(cnt_hbm.at[slot], cvec)
            cnt = jnp.max(cvec[...])
            nch = (cnt + (K - 1)) // K
            pltpu.sync_copy(lsrc_hbm.at[slot], sidx)
            pltpu.sync_copy(ldst_hbm.at[slot], didx)

            def g_desc(c, buf, sem):
                return pltpu.make_async_copy(hw_hbm.at[sidx.at[c]], buf, sem)

            def s_desc(c, buf, sem):
                return pltpu.make_async_copy(buf, agg.at[didx.at[c]], sem)

            @pl.when(nch > 0)
            def _():
                pltpu.async_copy(hw_hbm.at[sidx.at[0]], buf0, gs0)

            @pl.loop(0, NCAP, step=2)
            def _(c):
                @pl.when(jnp.logical_and(c > 0, c - 1 < nch))
                def _():
                    s_desc(c - 1, buf1, ss1).wait()

                @pl.when(c < nch)
                def _():
                    g_desc(c, buf0, gs0).wait()

                    @pl.when(c + 1 < nch)
                    def _():
                        pltpu.async_copy(hw_hbm.at[sidx.at[c + 1]], buf1, gs1)

                    pltpu.async_copy(buf0, agg.at[didx.at[c]], ss0, add=True)

                    @pl.when(c + 1 < nch)
                    def _():
                        g_desc(c + 1, buf1, gs1).wait()

                    s_desc(c, buf0, ss0).wait()

                    @pl.when(c + 2 < nch)
                    def _():
                        pltpu.async_copy(hw_hbm.at[sidx.at[c + 2]], buf0, gs0)

                    @pl.when(c + 1 < nch)
                    def _():
                        pltpu.async_copy(buf1, agg.at[didx.at[c + 1]], ss1,
                                         add=True)

            @pl.when(nch >= NCAP)
            def _():
                s_desc(NCAP - 1, buf1, ss1).wait()

        plsc.subcore_barrier()
        pltpu.sync_copy(agg.at[pl.ds(sid * ARPT, ARPT)],
                        out_hbm.at[q, pl.ds(sid * ARPT, ARPT)])

        plsc.subcore_barrier()


_msg_kernel = functools.partial(
    pl.kernel,
    out_type=jax.ShapeDtypeStruct((NR, AROWS, 2, 128), jnp.float32),
    mesh=_SC_MESH,
    compiler_params=_SC_CP,
    scratch_types=[
        pltpu.VMEM((NCAP, K), jnp.int32),
        pltpu.VMEM((NCAP, K), jnp.int32),
        pltpu.VMEM((16,), jnp.int32),
        pltpu.VMEM((K, 2, 128), jnp.float32),
        pltpu.VMEM((K, 2, 128), jnp.float32),
        pltpu.VMEM((8, 2, 128), jnp.float32),
        pltpu.VMEM_SHARED((AROWS, 2, 128), jnp.float32),
        pltpu.SemaphoreType.DMA,
        pltpu.SemaphoreType.DMA,
        pltpu.SemaphoreType.DMA,
        pltpu.SemaphoreType.DMA,
    ],
)(_msg_body)


def _dense_body(x_ref, w1_ref, b1_ref, wg_ref, deg_ref, hw_ref, dinv_ref):
    h = jnp.maximum(
        jnp.dot(x_ref[...], w1_ref[...],
                preferred_element_type=jnp.float32) + b1_ref[...], 0.0)
    hw = jnp.dot(h, wg_ref[...], preferred_element_type=jnp.float32)
    deg = jnp.sum(deg_ref[...], axis=0)
    dinv = jnp.where(deg > 0, lax.rsqrt(deg), 0.0)
    dinv_ref[...] = dinv[None, :]
    hw_ref[...] = hw * dinv[:, None]


def _final_body(p_ref, dinv_ref, bg_ref, wc_ref, bc_ref, o_ref):
    dinv = dinv_ref[0, :][:, None]
    agg = p_ref[0].reshape(BN, DIN)
    h2 = jnp.maximum(agg * dinv + bg_ref[...], 0.0)
    o_ref[...] = jnp.dot(h2, wc_ref[...],
                         preferred_element_type=jnp.float32) + bc_ref[...]


def kernel(X, edge_index, W1, b1, Wg, bg, Wc, bc):
    src = edge_index[0].astype(jnp.int32)
    dst = edge_index[1].astype(jnp.int32)
    loop_idx = jnp.arange(N, dtype=jnp.int32)
    pad_idx = jnp.full((E2P - E - N,), NP - 1, dtype=jnp.int32)
    srcf = jnp.concatenate([src, loop_idx, pad_idx]).reshape(NW, EPT)
    dstf = jnp.concatenate([dst, loop_idx, pad_idx]).reshape(NW, EPT)
    Xp = jnp.pad(X, ((0, NP - N), (0, 0)))

    degp, lsrc, ldst, cnts = _deg_kernel(dstf, srcf)

    hw2, dinv = pl.pallas_call(
        _dense_body,
        grid=(NP // BN,),
        in_specs=[
            pl.BlockSpec((BN, DIN), lambda i: (i, 0)),
            pl.BlockSpec((DIN, DIN), lambda i: (0, 0)),
            pl.BlockSpec((1, DIN), lambda i: (0, 0)),
            pl.BlockSpec((DIN, DIN), lambda i: (0, 0)),
            pl.BlockSpec((NW, BN), lambda i: (0, i)),
        ],
        out_specs=[
            pl.BlockSpec((BN, DIN), lambda i: (i, 0)),
            pl.BlockSpec((1, BN), lambda i: (0, i)),
        ],
        out_shape=[
            jax.ShapeDtypeStruct((NP, DIN), jnp.float32),
            jax.ShapeDtypeStruct((1, NP), jnp.float32),
        ],
    )(Xp, W1, b1.reshape(1, DIN), Wg, degp)

    p = _msg_kernel(hw2.reshape(NP, 2, 128),
                    lsrc.reshape(NW * NR, NCAP, K),
                    ldst.reshape(NW * NR, NCAP, K),
                    cnts.reshape(NW * NR, 16))

    out = pl.pallas_call(
        _final_body,
        grid=(NP // BN,),
        in_specs=[
            pl.BlockSpec((1, BN, 2, 128), lambda i: (i // 5, i % 5, 0, 0)),
            pl.BlockSpec((1, BN), lambda i: (0, i)),
            pl.BlockSpec((1, DIN), lambda i: (0, 0)),
            pl.BlockSpec((DIN, C), lambda i: (0, 0)),
            pl.BlockSpec((1, C), lambda i: (0, 0)),
        ],
        out_specs=pl.BlockSpec((BN, C), lambda i: (i, 0)),
        out_shape=jax.ShapeDtypeStruct((N, C), jnp.float32),
    )(p, dinv, bg.reshape(1, DIN), Wc, bc.reshape(1, C))

    return out


# speedup vs baseline: 1.6303x; 1.5252x over previous
import functools

import jax
import jax.numpy as jnp
from jax import lax
from jax.experimental import pallas as pl
from jax.experimental.pallas import tpu as pltpu
from jax.experimental.pallas import tpu_sc as plsc

N = 10000
NP = 10240
DIN = 256
DH = 128
C = 40
E = 160000

NC = 2
NS = 16
NW = NC * NS
K = 128
EPT = 5376
NCHUNK = EPT // K
E2P = NW * EPT
RPT = NP // NS

BN = 512

_SC_MESH = plsc.VectorSubcoreMesh(core_axis_name="c", subcore_axis_name="s")
_SC_CP = pltpu.CompilerParams(needs_layout_passes=False)


def _deg_body(dst_hbm, deg_hbm, dst_v, hist):
    cid = lax.axis_index("c")
    sid = lax.axis_index("s")
    wid = cid * NS + sid
    pltpu.sync_copy(dst_hbm.at[wid], dst_v)

    @pl.loop(0, NP, step=16)
    def _(i):
        hist[pl.ds(i, 16)] = jnp.zeros((16,), jnp.float32)

    ones = jnp.ones((16,), jnp.float32)

    @pl.loop(0, EPT, step=16)
    def _(e):
        plsc.addupdate_scatter(hist, [dst_v[pl.ds(e, 16)]], ones)

    pltpu.sync_copy(hist, deg_hbm.at[wid])


_deg_kernel = functools.partial(
    pl.kernel,
    out_type=jax.ShapeDtypeStruct((NW, NP), jnp.float32),
    mesh=_SC_MESH,
    compiler_params=_SC_CP,
    scratch_types=[
        pltpu.VMEM((EPT,), jnp.int32),
        pltpu.VMEM((NP,), jnp.float32),
    ],
)(_deg_body)


def _msg_body(hwa_hbm, hwb_hbm, srcp_hbm, dstp_hbm, outa_hbm, outb_hbm,
              src_v, dst_v, buf0, buf1, zbuf, agg, gs0, gs1, ss0, ss1):
    cid = lax.axis_index("c")
    sid = lax.axis_index("s")
    wid = cid * NS + sid
    pltpu.sync_copy(srcp_hbm.at[wid], src_v)
    pltpu.sync_copy(dstp_hbm.at[wid], dst_v)

    @pl.loop(0, 16)
    def _(r):
        @pl.loop(0, DH, step=16)
        def _(j):
            zbuf[r, pl.ds(j, 16)] = jnp.zeros((16,), jnp.float32)

    for hw_hbm, out_hbm in ((hwa_hbm, outa_hbm), (hwb_hbm, outb_hbm)):
        @pl.loop(0, RPT, step=16)
        def _(r):
            pltpu.sync_copy(zbuf, agg.at[pl.ds(sid * RPT + r, 16)])

        plsc.subcore_barrier()

        pltpu.async_copy(hw_hbm.at[src_v.at[0]], buf0, gs0)

        @pl.loop(0, NCHUNK, step=2)
        def _(c):
            pltpu.make_async_copy(hw_hbm.at[src_v.at[c]], buf0, gs0).wait()

            @pl.when(c > 0)
            def _():
                pltpu.make_async_copy(buf1, agg.at[dst_v.at[c - 1]],
                                      ss1).wait()

            pltpu.async_copy(hw_hbm.at[src_v.at[c + 1]], buf1, gs1)
            pltpu.async_copy(buf0, agg.at[dst_v.at[c]], ss0, add=True)
            pltpu.make_async_copy(hw_hbm.at[src_v.at[c + 1]], buf1, gs1).wait()
            pltpu.make_async_copy(buf0, agg.at[dst_v.at[c]], ss0).wait()

            @pl.when(c + 2 < NCHUNK)
            def _():
                pltpu.async_copy(hw_hbm.at[src_v.at[c + 2]], buf0, gs0)

            pltpu.async_copy(buf1, agg.at[dst_v.at[c + 1]], ss1, add=True)

        pltpu.make_async_copy(buf1, agg.at[dst_v.at[NCHUNK - 1]], ss1).wait()

        plsc.subcore_barrier()
        pltpu.sync_copy(agg.at[pl.ds(sid * RPT, RPT)],
                        out_hbm.at[cid, pl.ds(sid * RPT, RPT)])


_msg_kernel = functools.partial(
    pl.kernel,
    out_type=(jax.ShapeDtypeStruct((NC, NP, DH), jnp.float32),
              jax.ShapeDtypeStruct((NC, NP, DH), jnp.float32)),
    mesh=_SC_MESH,
    scratch_types=[
        pltpu.VMEM((NCHUNK, K), jnp.int32),
        pltpu.VMEM((NCHUNK, K), jnp.int32),
        pltpu.VMEM((K, DH), jnp.float32),
        pltpu.VMEM((K, DH), jnp.float32),
        pltpu.VMEM((16, DH), jnp.float32),
        pltpu.VMEM_SHARED((NP, DH), jnp.float32),
        pltpu.SemaphoreType.DMA,
        pltpu.SemaphoreType.DMA,
        pltpu.SemaphoreType.DMA,
        pltpu.SemaphoreType.DMA,
    ],
)(_msg_body)


def _dense_body(x_ref, w1_ref, b1_ref, wg_ref, deg_ref,
                hwa_ref, hwb_ref, dinv_ref):
    h = jnp.maximum(
        jnp.dot(x_ref[...], w1_ref[...], preferred_element_type=jnp.float32) + b1_ref[...], 0.0)
    hw = jnp.dot(h, wg_ref[...], preferred_element_type=jnp.float32)
    deg = jnp.sum(deg_ref[...], axis=0)
    dinv = jnp.where(deg > 0, lax.rsqrt(deg), 0.0)
    dinv_ref[...] = dinv[None, :]
    hwa_ref[...] = hw[:, :DH] * dinv[:, None]
    hwb_ref[...] = hw[:, DH:] * dinv[:, None]


def _final_body(pa_ref, pb_ref, dinv_ref, bg_ref, wc_ref, bc_ref, o_ref):
    dinv = dinv_ref[0, :][:, None]
    hl = jnp.maximum((pa_ref[0] + pa_ref[1]) * dinv + bg_ref[:, :DH], 0.0)
    hr = jnp.maximum((pb_ref[0] + pb_ref[1]) * dinv + bg_ref[:, DH:], 0.0)
    h2 = jnp.concatenate([hl, hr], axis=1)
    o_ref[...] = jnp.dot(h2, wc_ref[...], preferred_element_type=jnp.float32) + bc_ref[...]


def kernel(X, edge_index, W1, b1, Wg, bg, Wc, bc):
    src = edge_index[0].astype(jnp.int32)
    dst = edge_index[1].astype(jnp.int32)
    loop_idx = jnp.arange(N, dtype=jnp.int32)
    pad_idx = jnp.full((E2P - E - N,), NP - 1, dtype=jnp.int32)
    srcp = jnp.concatenate([src, loop_idx, pad_idx]).reshape(NW, NCHUNK, K)
    dst_flat = jnp.concatenate([dst, loop_idx, pad_idx])
    dstp = dst_flat.reshape(NW, NCHUNK, K)
    dstf = dst_flat.reshape(NW, EPT)
    Xp = jnp.pad(X, ((0, NP - N), (0, 0)))

    degp = _deg_kernel(dstf)

    hwa, hwb, dinv = pl.pallas_call(
        _dense_body,
        grid=(NP // BN,),
        in_specs=[
            pl.BlockSpec((BN, DIN), lambda i: (i, 0)),
            pl.BlockSpec((DIN, DIN), lambda i: (0, 0)),
            pl.BlockSpec((1, DIN), lambda i: (0, 0)),
            pl.BlockSpec((DIN, DIN), lambda i: (0, 0)),
            pl.BlockSpec((NW, BN), lambda i: (0, i)),
        ],
        out_specs=[
            pl.BlockSpec((BN, DH), lambda i: (i, 0)),
            pl.BlockSpec((BN, DH), lambda i: (i, 0)),
            pl.BlockSpec((1, BN), lambda i: (0, i)),
        ],
        out_shape=[
            jax.ShapeDtypeStruct((NP, DH), jnp.float32),
            jax.ShapeDtypeStruct((NP, DH), jnp.float32),
            jax.ShapeDtypeStruct((1, NP), jnp.float32),
        ],
    )(Xp, W1, b1.reshape(1, DIN), Wg, degp)

    pa, pb = _msg_kernel(hwa, hwb, srcp, dstp)

    out = pl.pallas_call(
        _final_body,
        grid=(NP // BN,),
        in_specs=[
            pl.BlockSpec((NC, BN, DH), lambda i: (0, i, 0)),
            pl.BlockSpec((NC, BN, DH), lambda i: (0, i, 0)),
            pl.BlockSpec((1, BN), lambda i: (0, i)),
            pl.BlockSpec((1, DIN), lambda i: (0, 0)),
            pl.BlockSpec((DIN, C), lambda i: (0, 0)),
            pl.BlockSpec((1, C), lambda i: (0, 0)),
        ],
        out_specs=pl.BlockSpec((BN, C), lambda i: (i, 0)),
        out_shape=jax.ShapeDtypeStruct((N, C), jnp.float32),
    )(pa, pb, dinv, bg.reshape(1, DIN), Wc, bc.reshape(1, C))

    return out
